# baseline (speedup 1.0000x reference)
import functools

import jax
import jax.numpy as jnp
from jax import lax
from jax.experimental import pallas as pl
from jax.experimental.pallas import tpu as pltpu
from jax.experimental.pallas import tpu_sc as plsc

N = 10000
K = 27
E_PER = 5925
E = K * E_PER
E_PAD = 163840
N_SUB = 16
E_SUB = E_PAD // 32
ACC_ROWS = 10240
BN2 = 200
BND = 1000

_SC_CFG = {64: (64, 1, 128, 4, 2), 128: (128, 1, 64, 2, 1),
           256: (128, 2, 64, 2, 1)}


def _act_cat(part_refs, aff_refs, relu):
    pieces = []
    for pr, ar in zip(part_refs, aff_refs):
        v = pr[0] if pr.shape[0] == 1 else pr[0] + pr[1]
        v = v * ar[0][None, :] + ar[1][None, :]
        if relu:
            v = jnp.maximum(v, 0.0)
        pieces.append(v)
    return pieces[0] if len(pieces) == 1 else jnp.concatenate(pieces, -1)


def _mm_conv(parts, affs, w, relu, p_split):
    cin, cout = w.shape[1], w.shape[2]
    cw = cout // p_split
    ncols = p_split * K * cw
    wr = (w.transpose(1, 0, 2)
          .reshape(cin, K, p_split, cw)
          .transpose(0, 2, 1, 3)
          .reshape(cin, ncols)
          .astype(jnp.bfloat16))
    npart = len(parts)

    def body(*refs):
        part_refs = refs[:npart]
        aff_refs = refs[npart:2 * npart]
        w_ref = refs[2 * npart]
        h_ref = refs[2 * npart + 1]
        xcat = _act_cat(part_refs, aff_refs, relu).astype(jnp.bfloat16)
        h_ref[...] = jnp.dot(xcat, w_ref[...],
                             preferred_element_type=jnp.float32)

    out = pl.pallas_call(
        body,
        grid=(N // BN2,),
        in_specs=(
            [pl.BlockSpec((p.shape[0], BN2, p.shape[2]),
                          lambda nb: (0, nb, 0)) for p in parts]
            + [pl.BlockSpec(a.shape, lambda nb: (0, 0)) for a in affs]
            + [pl.BlockSpec((cin, ncols), lambda nb: (0, 0))]
        ),
        out_specs=pl.BlockSpec((BN2, ncols), lambda nb: (nb, 0)),
        out_shape=jax.ShapeDtypeStruct((N, ncols), jnp.float32),
    )(*parts, *affs, wr)
    return out.reshape(N * p_split * K, cw)


def _stats(parts, g, b):
    npart = len(parts)
    cws = [p.shape[2] for p in parts]
    offs = [sum(cws[:i]) for i in range(npart)]

    def body(*refs):
        part_refs = refs[:npart]
        g_refs = refs[npart:2 * npart]
        b_refs = refs[2 * npart:3 * npart]
        o_refs = refs[3 * npart:]
        for pr, gr, br, orf in zip(part_refs, g_refs, b_refs, o_refs):
            yb = (pr[0] if pr.shape[0] == 1 else pr[0] + pr[1])[:N]
            mean = jnp.mean(yb, axis=0)
            var = jnp.mean(yb * yb, axis=0) - mean * mean
            scale = gr[...] * lax.rsqrt(var + 1e-5)
            orf[...] = jnp.stack([scale, br[...] - mean * scale], axis=0)

    return pl.pallas_call(
        body,
        out_shape=[jax.ShapeDtypeStruct((2, cw), jnp.float32) for cw in cws],
    )(*parts, *[g[o:o + cw] for o, cw in zip(offs, cws)],
      *[b[o:o + cw] for o, cw in zip(offs, cws)])


def _mm_dense(parts, affs, w):
    cout = w.shape[1]
    npart = len(parts)

    def body(*refs):
        part_refs = refs[:npart]
        aff_refs = refs[npart:2 * npart]
        w_ref = refs[2 * npart]
        o_ref = refs[2 * npart + 1]
        xcat = _act_cat(part_refs, aff_refs, True)
        o_ref[...] = jnp.dot(xcat, w_ref[...],
                             preferred_element_type=jnp.float32)

    return pl.pallas_call(
        body,
        grid=(N // BND,),
        in_specs=(
            [pl.BlockSpec((p.shape[0], BND, p.shape[2]),
                          lambda nb: (0, nb, 0)) for p in parts]
            + [pl.BlockSpec(a.shape, lambda nb: (0, 0)) for a in affs]
            + [pl.BlockSpec(w.shape, lambda nb: (0, 0))]
        ),
        out_specs=pl.BlockSpec((BND, cout), lambda nb: (nb, 0)),
        out_shape=jax.ShapeDtypeStruct((N, cout), jnp.float32),
    )(*parts, *affs, w)


@functools.lru_cache(maxsize=None)
def _make_sc_conv(cw, chunk, nbuf, la):
    mesh = plsc.VectorSubcoreMesh(core_axis_name="c", subcore_axis_name="s")
    nch = E_SUB // chunk

    @functools.partial(
        pl.kernel,
        mesh=mesh,
        compiler_params=pltpu.CompilerParams(use_tc_tiling_on_sc=False),
        out_type=jax.ShapeDtypeStruct((2, ACC_ROWS, cw), jnp.float32),
        scratch_types=[
            pltpu.VMEM((nch, chunk), jnp.int32),
            pltpu.VMEM((nch, chunk), jnp.int32),
            [pltpu.VMEM((chunk, cw), jnp.float32) for _ in range(nbuf)],
            pltpu.VMEM_SHARED((ACC_ROWS, cw), jnp.float32),
            [pltpu.SemaphoreType.DMA for _ in range(nbuf)],
            [pltpu.SemaphoreType.DMA for _ in range(nbuf)],
        ],
    )
    def sc_conv(h_hbm, gidx_hbm, dst_hbm, y_hbm, gv, dv, rows, acc,
                gsems, ssems):
        zbuf = rows[0]
        c = lax.axis_index("c")
        s = lax.axis_index("s")
        zvec = jnp.zeros((16,), jnp.float32)

        def zrow(r, carry):
            for i in range(cw // 16):
                zbuf[r, pl.ds(i * 16, 16)] = zvec
            return carry

        lax.fori_loop(0, chunk, zrow, 0)
        for t in range(640 // chunk):
            pltpu.sync_copy(zbuf, acc.at[pl.ds(s * 640 + t * chunk, chunk)])
        pltpu.sync_copy(gidx_hbm.at[c, s], gv)
        pltpu.sync_copy(dst_hbm.at[c, s], dv)
        plsc.subcore_barrier()

        for b in range(la):
            pltpu.async_copy(h_hbm.at[gv.at[b]], rows[b], gsems[b])

        def step(g, carry):
            for i in range(nbuf):
                j = g * nbuf + i
                b = i
                pltpu.make_async_copy(h_hbm.at[gv.at[j]], rows[b],
                                      gsems[b]).wait()
                pltpu.async_copy(rows[b], acc.at[dv.at[j]], ssems[b],
                                 add=True)
                jd = j - la
                bd = (i - la) % nbuf

                @pl.when(jd >= 0)
                def _():
                    pltpu.make_async_copy(
                        rows[bd], acc.at[dv.at[jd]], ssems[bd]).wait()

                jn = j + la
                bn = (i + la) % nbuf

                @pl.when(jn < nch)
                def _():
                    pltpu.async_copy(h_hbm.at[gv.at[jn]], rows[bn], gsems[bn])
            return carry

        lax.fori_loop(0, nch // nbuf, step, 0)
        for i in range(la):
            j = nch - la + i
            pltpu.make_async_copy(rows[j % nbuf], acc.at[dv.at[j]],
                                  ssems[j % nbuf]).wait()
        plsc.subcore_barrier()
        pltpu.sync_copy(acc.at[pl.ds(s * 640, 640)],
                        y_hbm.at[c, pl.ds(s * 640, 640)])

    return sc_conv


def _sc_apply(h_flat, gidx, dst, cw, chunk, nbuf, la):
    return _make_sc_conv(cw, chunk, nbuf, la)(h_flat, gidx, dst)


def _conv(parts, affs, relu, w, gidx_all, dst_all):
    cout = w.shape[2]
    cw, p_split, chunk, nbuf, la = _SC_CFG[cout]
    h_flat = _mm_conv(parts, affs, w, relu, p_split)
    return [_sc_apply(h_flat, gidx_all[cout][p], dst_all[chunk],
                      cw, chunk, nbuf, la)
            for p in range(p_split)]


def _block(parts, affs, relu_in, p, gidx_all, dst_all):
    y1 = _conv(parts, affs, relu_in, p["W1"], gidx_all, dst_all)
    aff1 = _stats(y1, p["g1"], p["b1"])
    y2 = _conv(y1, aff1, True, p["W2"], gidx_all, dst_all)
    aff2 = _stats(y2, p["g2"], p["b2"])
    return y2, aff2


def kernel(x, edge_index, params):
    src = edge_index[0].astype(jnp.int32)
    dst = edge_index[1].astype(jnp.int32)
    k_of = jnp.arange(E, dtype=jnp.int32) // E_PER
    zpad = jnp.zeros((E_PAD - E,), jnp.int32)
    gidx_all = {}
    for cout, (cw, p_split, chunk, _, _) in _SC_CFG.items():
        base = jnp.concatenate([src * (p_split * K) + k_of, zpad])
        gidx_all[cout] = [
            (base + p * K).reshape(2, N_SUB, E_SUB // chunk, chunk)
            for p in range(p_split)
        ]
    dstp = jnp.concatenate([dst, jnp.full((E_PAD - E,), N, jnp.int32)])
    dst_all = {ch: dstp.reshape(2, N_SUB, E_SUB // ch, ch) for ch in (64, 128)}

    x0 = [x.reshape(1, N, 128)]
    one = jnp.ones((128,), jnp.float32)
    aff0 = [jnp.stack([one, jnp.zeros_like(one)], axis=0)]

    y_e0, aff_e0 = _block(x0, aff0, False, params["enc0"], gidx_all, dst_all)
    y_e1, aff_e1 = _block(y_e0, aff_e0, True, params["enc1"], gidx_all,
                          dst_all)
    y_bt, aff_bt = _block(y_e1, aff_e1, True, params["bottleneck"], gidx_all,
                          dst_all)

    d0 = _mm_dense(y_bt + y_e1, aff_bt + aff_e1, params["dec0"]["Wf"])
    d0p = [d0.reshape(1, N, d0.shape[1])]
    aff_d0 = _stats(d0p, params["dec0"]["g"], params["dec0"]["b"])
    d1 = _mm_dense(d0p + y_e0, aff_d0 + aff_e0, params["dec1"]["Wf"])
    d1p = [d1.reshape(1, N, d1.shape[1])]
    aff_d1 = _stats(d1p, params["dec1"]["g"], params["dec1"]["b"])
    return _mm_dense(d1p, aff_d1, params["final_W"])
